# Initial kernel scaffold; baseline (speedup 1.0000x reference)
import jax
import jax.numpy as jnp
from jax.experimental import pallas as pl


def kernel(x, W_enc, b_enc):
    raise NotImplementedError("write your pallas kernel here")


# speedup vs baseline: 7.5259x; 7.5259x over previous
import functools

import jax
import jax.numpy as jnp
from jax.experimental import pallas as pl

_K = 64


def _enc_block(x_ref, w_ref, b_ref, o_ref, *, k):
    y = jnp.dot(x_ref[...], w_ref[...], preferred_element_type=jnp.float32)
    a = jnp.maximum(y + b_ref[...], 0.0)
    bits = jax.lax.bitcast_convert_type(a, jnp.int32)
    t = jnp.zeros((a.shape[0], 1), jnp.int32)
    for bit in range(30, -1, -1):
        cand = t | (1 << bit)
        cnt = jnp.sum((bits >= cand).astype(jnp.int32), axis=1, keepdims=True)
        t = jnp.where(cnt >= k, cand, t)
    thr = jax.lax.bitcast_convert_type(t, jnp.float32)
    o_ref[...] = jnp.where(a >= thr, a, 0.0)


def kernel(x, W_enc, b_enc):
    n_tokens, d_in = x.shape
    n_features = W_enc.shape[1]
    bm = 512
    b2 = b_enc.reshape(1, n_features)
    return pl.pallas_call(
        functools.partial(_enc_block, k=_K),
        grid=(n_tokens // bm,),
        in_specs=[
            pl.BlockSpec((bm, d_in), lambda i: (i, 0)),
            pl.BlockSpec((d_in, n_features), lambda i: (0, 0)),
            pl.BlockSpec((1, n_features), lambda i: (0, 0)),
        ],
        out_specs=pl.BlockSpec((bm, n_features), lambda i: (i, 0)),
        out_shape=jax.ShapeDtypeStruct((n_tokens, n_features), jnp.float32),
    )(x, W_enc, b2)


# speedup vs baseline: 11.6051x; 1.5420x over previous
import functools

import jax
import jax.numpy as jnp
from jax.experimental import pallas as pl

_K = 64


def _rowcount(m16):
    n = m16.shape[1]
    part = m16[:, 0:128]
    for j in range(128, n, 128):
        part = part + m16[:, j:j + 128]
    return jnp.sum(part.astype(jnp.float32), axis=1, keepdims=True)


def _enc_block(x_ref, w_ref, b_ref, o_ref, *, k):
    y = jnp.dot(x_ref[...], w_ref[...], preferred_element_type=jnp.float32)
    a = jnp.maximum(y + b_ref[...], 0.0)
    bits = jax.lax.bitcast_convert_type(a, jnp.int32)
    rows = a.shape[0]

    key16 = (bits >> 16).astype(jnp.int16)
    low16 = ((bits & 0xFFFF) ^ 0x8000).astype(jnp.int16)

    p = jnp.zeros((rows, 1), jnp.int32)
    for bit in range(14, -1, -1):
        cand = p | (1 << bit)
        cand16 = cand.astype(jnp.int16)
        cnt = _rowcount((key16 >= cand16).astype(jnp.int16))
        p = jnp.where(cnt >= float(k), cand, p)

    p16 = p.astype(jnp.int16)
    g = _rowcount((key16 > p16).astype(jnp.int16))
    need = float(k) - g
    eq16 = (key16 == p16).astype(jnp.int16)
    s = jnp.zeros((rows, 1), jnp.int32)
    for bit in range(15, -1, -1):
        cand_u = s | (1 << bit)
        cand_s = (cand_u ^ 0x8000).astype(jnp.int16)
        m = jnp.where(low16 >= cand_s, eq16, jnp.int16(0))
        cnt = _rowcount(m)
        s = jnp.where(cnt >= need, cand_u, s)

    vbits = (p << 16) | s
    o_ref[...] = jnp.where(bits >= vbits, a, 0.0)


def kernel(x, W_enc, b_enc):
    n_tokens, d_in = x.shape
    n_features = W_enc.shape[1]
    bm = 512
    b2 = b_enc.reshape(1, n_features)
    return pl.pallas_call(
        functools.partial(_enc_block, k=_K),
        grid=(n_tokens // bm,),
        in_specs=[
            pl.BlockSpec((bm, d_in), lambda i: (i, 0)),
            pl.BlockSpec((d_in, n_features), lambda i: (0, 0)),
            pl.BlockSpec((1, n_features), lambda i: (0, 0)),
        ],
        out_specs=pl.BlockSpec((bm, n_features), lambda i: (i, 0)),
        out_shape=jax.ShapeDtypeStruct((n_tokens, n_features), jnp.float32),
    )(x, W_enc, b2)


# speedup vs baseline: 11.8626x; 1.0222x over previous
import functools

import jax
import jax.numpy as jnp
from jax.experimental import pallas as pl

_K = 64


def _rowcount(m16):
    n = m16.shape[1]
    part = m16[:, 0:128]
    for j in range(128, n, 128):
        part = part + m16[:, j:j + 128]
    return jnp.sum(part.astype(jnp.float32), axis=1, keepdims=True)


def _select16(bits, k):
    rows = bits.shape[0]
    key16 = (bits >> 16).astype(jnp.int16)
    low16 = (bits ^ 0x8000).astype(jnp.int16)

    p = jnp.zeros((rows, 1), jnp.int32)
    for bit in range(14, -1, -1):
        cand = p | (1 << bit)
        cand16 = cand.astype(jnp.int16)
        cnt = _rowcount((key16 >= cand16).astype(jnp.int16))
        p = jnp.where(cnt >= float(k), cand, p)

    p16 = p.astype(jnp.int16)
    g = _rowcount((key16 > p16).astype(jnp.int16))
    need = float(k) - g
    z = jnp.where(key16 == p16, low16, jnp.int16(-32768))
    s = jnp.zeros((rows, 1), jnp.int32)
    for bit in range(15, -1, -1):
        cand_u = s | (1 << bit)
        cand_s = (cand_u ^ 0x8000).astype(jnp.int16)
        cnt = _rowcount((z >= cand_s).astype(jnp.int16))
        s = jnp.where(cnt >= need, cand_u, s)

    return (p << 16) | s


def _enc_block(x_ref, w_ref, b_ref, o_ref, *, k):
    y = jnp.dot(x_ref[...], w_ref[...], preferred_element_type=jnp.float32)
    a = jnp.maximum(y + b_ref[...], 0.0)
    bits = jax.lax.bitcast_convert_type(a, jnp.int32)
    vbits = _select16(bits, k)
    o_ref[...] = jnp.where(bits >= vbits, a, 0.0)


def kernel(x, W_enc, b_enc):
    n_tokens, d_in = x.shape
    n_features = W_enc.shape[1]
    bm = 1024
    b2 = b_enc.reshape(1, n_features)
    return pl.pallas_call(
        functools.partial(_enc_block, k=_K),
        grid=(n_tokens // bm,),
        in_specs=[
            pl.BlockSpec((bm, d_in), lambda i: (i, 0)),
            pl.BlockSpec((d_in, n_features), lambda i: (0, 0)),
            pl.BlockSpec((1, n_features), lambda i: (0, 0)),
        ],
        out_specs=pl.BlockSpec((bm, n_features), lambda i: (i, 0)),
        out_shape=jax.ShapeDtypeStruct((n_tokens, n_features), jnp.float32),
    )(x, W_enc, b2)
